# Initial kernel scaffold; baseline (speedup 1.0000x reference)
import jax
import jax.numpy as jnp
from jax.experimental import pallas as pl


def kernel(base_indices, reservoir_encoded, embedding):
    raise NotImplementedError("write your pallas kernel here")


# speedup vs baseline: 27.8621x; 27.8621x over previous
import jax
import jax.numpy as jnp
from jax import lax
from jax.experimental import pallas as pl
from jax.experimental.pallas import tpu as pltpu
from jax.experimental.pallas import tpu_sc as plsc

_VOCAB = 30522
_NTOK = 100000
_RES = 8
_FEAT = 32
_FROZEN = 0
_BATCH = 16384
_FIELDS = 26

_NC = 2
_NS = 16
_NW = _NC * _NS

_S1_CHUNK_T = 128
_NTOK_PAD = ((_NTOK + _S1_CHUNK_T - 1) // _S1_CHUNK_T) * _S1_CHUNK_T
_S1_NCH = _NTOK_PAD // _S1_CHUNK_T
_S1_ITERS = (_S1_NCH + _NW - 1) // _NW

_NLOOK = _BATCH * _FIELDS
_S2_ROWS = _NLOOK // 128
_S2_K = 8
_S2_ITERS = _S2_ROWS // (_NW * _S2_K)


def _wid():
    return lax.axis_index("s") * _NC + lax.axis_index("c")


def _pool_body(res2d, emb, pooled, ids_v, rows_v, out_v, sem):
    w = _wid()

    def chunk_body(it, carry):
        c = w + it * _NW

        @pl.when(c < _S1_NCH)
        def _():
            pltpu.sync_copy(res2d.at[pl.ds(c * _S2_K, _S2_K)], ids_v)
            descs = [
                pltpu.async_copy(emb.at[ids_v.at[j]], rows_v.at[j], sem)
                for j in range(_S2_K)
            ]
            for d in descs:
                d.wait()

            def tok_body(t, _):
                j = t // 16
                p = (t % 16) * 8
                for h in range(2):
                    acc = rows_v[j, p, pl.ds(16 * h, 16)]
                    for r in range(1, _RES):
                        acc = acc + rows_v[j, p + r, pl.ds(16 * h, 16)]
                    out_v[t, pl.ds(16 * h, 16)] = acc
                return 0

            lax.fori_loop(0, _S1_CHUNK_T, tok_body, 0)
            pltpu.sync_copy(out_v, pooled.at[pl.ds(c * _S1_CHUNK_T, _S1_CHUNK_T)])

        return carry

    lax.fori_loop(0, _S1_ITERS, chunk_body, 0)


def _lookup_body(base2d, pooled, out3d, ids_v, rows_v, sem):
    w = _wid()

    def chunk_body(it, carry):
        r0 = (w * _S2_ITERS + it) * _S2_K
        pltpu.sync_copy(base2d.at[pl.ds(r0, _S2_K)], ids_v)
        descs = [
            pltpu.async_copy(pooled.at[ids_v.at[j]], rows_v.at[j], sem)
            for j in range(_S2_K)
        ]
        for d in descs:
            d.wait()
        pltpu.sync_copy(rows_v, out3d.at[pl.ds(r0, _S2_K)])
        return carry

    lax.fori_loop(0, _S2_ITERS, chunk_body, 0)


def kernel(base_indices, reservoir_encoded, embedding):
    emb0 = embedding.at[_FROZEN].set(0.0)
    res_flat = reservoir_encoded.reshape(-1)
    res_pad = jnp.concatenate(
        [res_flat, jnp.zeros(((_NTOK_PAD - _NTOK) * _RES,), jnp.int32)]
    )
    res2d = res_pad.reshape(_NTOK_PAD * _RES // 128, 128)
    base2d = base_indices.reshape(_S2_ROWS, 128)

    mesh = plsc.VectorSubcoreMesh(core_axis_name="c", subcore_axis_name="s")

    params = pltpu.CompilerParams(use_tc_tiling_on_sc=False)

    pooled = pl.kernel(
        _pool_body,
        out_type=jax.ShapeDtypeStruct((_NTOK_PAD, _FEAT), jnp.float32),
        mesh=mesh,
        compiler_params=params,
        scratch_types=[
            pltpu.VMEM((_S2_K, 128), jnp.int32),
            pltpu.VMEM((_S2_K, 128, _FEAT), jnp.float32),
            pltpu.VMEM((_S1_CHUNK_T, _FEAT), jnp.float32),
            pltpu.SemaphoreType.DMA,
        ],
    )(res2d, emb0)

    out3d = pl.kernel(
        _lookup_body,
        out_type=jax.ShapeDtypeStruct((_S2_ROWS, 128, _FEAT), jnp.float32),
        mesh=mesh,
        compiler_params=params,
        scratch_types=[
            pltpu.VMEM((_S2_K, 128), jnp.int32),
            pltpu.VMEM((_S2_K, 128, _FEAT), jnp.float32),
            pltpu.SemaphoreType.DMA,
        ],
    )(base2d, pooled)

    return out3d.reshape(_BATCH, _FIELDS, _FEAT)


# speedup vs baseline: 28.5209x; 1.0236x over previous
import jax
import jax.numpy as jnp
from jax import lax
from jax.experimental import pallas as pl
from jax.experimental.pallas import tpu as pltpu
from jax.experimental.pallas import tpu_sc as plsc

_VOCAB = 30522
_NTOK = 100000
_RES = 8
_FEAT = 32
_FROZEN = 0
_BATCH = 16384
_FIELDS = 26

_NC = 2
_NS = 16
_NW = _NC * _NS

_S1_T = 128
_S1_NCH = (_NTOK + _S1_T - 1) // _S1_T
_S1_LAST_T0 = _NTOK - _S1_T
_S1_ITERS = (_S1_NCH + _NW - 1) // _NW
_S1_STREAMS = _S1_T * _RES // 128

_S2_B = 64
_S2_PER_W = _BATCH // _NW
_S2_ITERS = _S2_PER_W // _S2_B


def _wid():
    return lax.axis_index("s") * _NC + lax.axis_index("c")


def _pool_body(res_hbm, emb, pooled, ids_v, rows_v, out_v, sem):
    w = _wid()

    def chunk_body(it, carry):
        c = w + it * _NW

        @pl.when(c < _S1_NCH)
        def _():
            t0 = lax.min(c * _S1_T, _S1_LAST_T0)
            pltpu.sync_copy(res_hbm.at[pl.ds(t0 * _RES, _S1_T * _RES)], ids_v)
            descs = [
                pltpu.async_copy(
                    emb.at[ids_v.at[pl.ds(128 * j, 128)]],
                    rows_v.at[pl.ds(128 * j, 128)],
                    sem,
                )
                for j in range(_S1_STREAMS)
            ]
            for d in descs:
                d.wait()

            def tok_body(t, _):
                q = t * _RES
                for h in range(2):
                    acc = rows_v[q, pl.ds(16 * h, 16)]
                    for r in range(1, _RES):
                        acc = acc + rows_v[q + r, pl.ds(16 * h, 16)]
                    out_v[t, pl.ds(16 * h, 16)] = acc
                return 0

            lax.fori_loop(0, _S1_T, tok_body, 0)
            pltpu.sync_copy(out_v, pooled.at[pl.ds(t0, _S1_T)])

        return carry

    lax.fori_loop(0, _S1_ITERS, chunk_body, 0)


def _lookup_body(base_hbm, pooled, out_hbm, ids_v, rows_v, sem):
    w = _wid()

    def chunk_body(it, carry):
        b0 = w * _S2_PER_W + it * _S2_B
        pltpu.sync_copy(base_hbm.at[pl.ds(b0, _S2_B)], ids_v)

        def fire(b, carry2):
            pltpu.async_copy(pooled.at[ids_v.at[b]], rows_v.at[b], sem)
            return carry2

        lax.fori_loop(0, _S2_B, fire, 0)

        def drain(b, carry2):
            pltpu.make_async_copy(pooled.at[ids_v.at[b]], rows_v.at[b], sem).wait()
            return carry2

        lax.fori_loop(0, _S2_B, drain, 0)
        pltpu.sync_copy(rows_v, out_hbm.at[pl.ds(b0, _S2_B)])
        return carry

    lax.fori_loop(0, _S2_ITERS, chunk_body, 0)


def kernel(base_indices, reservoir_encoded, embedding):
    emb0 = embedding.at[_FROZEN].set(0.0)
    res1d = reservoir_encoded.reshape(-1)

    mesh = plsc.VectorSubcoreMesh(core_axis_name="c", subcore_axis_name="s")
    params = pltpu.CompilerParams(use_tc_tiling_on_sc=False)

    pooled = pl.kernel(
        _pool_body,
        out_type=jax.ShapeDtypeStruct((_NTOK, _FEAT), jnp.float32),
        mesh=mesh,
        compiler_params=params,
        scratch_types=[
            pltpu.VMEM((_S1_T * _RES,), jnp.int32),
            pltpu.VMEM((_S1_T * _RES, _FEAT), jnp.float32),
            pltpu.VMEM((_S1_T, _FEAT), jnp.float32),
            pltpu.SemaphoreType.DMA,
        ],
    )(res1d, emb0)

    out = pl.kernel(
        _lookup_body,
        out_type=jax.ShapeDtypeStruct((_BATCH, _FIELDS, _FEAT), jnp.float32),
        mesh=mesh,
        compiler_params=params,
        scratch_types=[
            pltpu.VMEM((_S2_B, _FIELDS), jnp.int32),
            pltpu.VMEM((_S2_B, _FIELDS, _FEAT), jnp.float32),
            pltpu.SemaphoreType.DMA,
        ],
    )(base_indices, pooled)

    return out
